# baseline (speedup 1.0000x reference)
import jax
import jax.numpy as jnp
import numpy as np
from jax import lax
from jax.experimental import pallas as pl
from jax.experimental.pallas import tpu as pltpu
from jax.experimental.pallas import tpu_sc as plsc

NC = 2
NS = 16
LN = 16
NW = NC * NS

B, C, H, W = 8, 96, 224, 224
N = H * W
R = B * C
RPW = R // NW
STR = N // LN
CHK = 448
NWIN = STR // CHK
FW = 1792
NFW = N // FW

MINI32 = np.int32(-2147483648)


def _sort_body(x_hbm, sv_hbm, rank_hbm, ords_hbm, keys, ordout, ordwin, hist,
               fwin):
    cid = lax.axis_index("c")
    sid = lax.axis_index("s")
    wid = sid * NC + cid
    iota = lax.iota(jnp.int32, LN)
    lane_base = iota * STR
    ones = jnp.ones((LN,), jnp.int32)

    def digit_at(ordv, shift):
        k = plsc.load_gather(keys, [ordv])
        d = (k >> shift) & 255
        return (d << 4) | iota

    def do_row(i, carry0):
        r = wid * RPW + i
        pltpu.sync_copy(x_hbm.at[r], keys)

        def tf_body(t, _):
            k = keys[pl.ds(t * LN, LN)]
            keys[pl.ds(t * LN, LN)] = k ^ ((k >> 31) | MINI32)
            return 0

        lax.fori_loop(0, STR, tf_body, 0)

        for p in range(4):
            shift = 8 * p

            def z_body(h, _):
                hist[pl.ds(h * LN, LN)] = jnp.zeros((LN,), jnp.int32)
                return 0

            lax.fori_loop(0, 256, z_body, 0)

            if p == 0:
                def c_body(t, _):
                    hidx = digit_at(lane_base + t, shift)
                    plsc.addupdate_scatter(hist, [hidx], ones)
                    return 0

                lax.fori_loop(0, STR, c_body, 0)
            else:
                def cw_body(w, _):
                    pltpu.sync_copy(
                        ords_hbm.at[r, :, pl.ds(w * CHK, CHK)], ordwin)

                    def ci_body(tt, _):
                        ordv = plsc.load_gather(ordwin, [iota, iota * 0 + tt])
                        hidx = digit_at(ordv, shift)
                        plsc.addupdate_scatter(hist, [hidx], ones)
                        return 0

                    lax.fori_loop(0, CHK, ci_body, 0)
                    return 0

                lax.fori_loop(0, NWIN, cw_body, 0)

            def pf_body(h, carry):
                v = hist[pl.ds(h * LN, LN)]
                inc = plsc.cumsum(v)
                hist[pl.ds(h * LN, LN)] = inc - v + carry
                return carry + jnp.sum(v)

            lax.fori_loop(0, 256, pf_body, jnp.int32(0))

            if p == 0:
                def s_body(t, _):
                    ordv = lane_base + t
                    hidx = digit_at(ordv, shift)
                    pos = plsc.load_gather(hist, [hidx])
                    plsc.store_scatter(ordout, [pos], ordv)
                    plsc.store_scatter(hist, [hidx], pos + 1)
                    return 0

                lax.fori_loop(0, STR, s_body, 0)
            else:
                def sw_body(w, _):
                    pltpu.sync_copy(
                        ords_hbm.at[r, :, pl.ds(w * CHK, CHK)], ordwin)

                    def si_body(tt, _):
                        ordv = plsc.load_gather(ordwin, [iota, iota * 0 + tt])
                        hidx = digit_at(ordv, shift)
                        pos = plsc.load_gather(hist, [hidx])
                        plsc.store_scatter(ordout, [pos], ordv)
                        plsc.store_scatter(hist, [hidx], pos + 1)
                        return 0

                    lax.fori_loop(0, CHK, si_body, 0)
                    return 0

                lax.fori_loop(0, NWIN, sw_body, 0)

            if p < 3:
                for l in range(LN):
                    pltpu.sync_copy(ordout.at[pl.ds(l * STR, STR)],
                                    ords_hbm.at[r, l])

        def f_body(w, _):
            def fi_body(tt, _):
                ordv = ordout[pl.ds(w * FW + tt * LN, LN)]
                k = plsc.load_gather(keys, [ordv])
                k = k ^ (jnp.bitwise_not(k >> 31) | MINI32)
                fwin[pl.ds(tt * LN, LN)] = plsc.bitcast(k, jnp.float32)
                return 0

            lax.fori_loop(0, FW // LN, fi_body, 0)
            pltpu.sync_copy(fwin, sv_hbm.at[r, pl.ds(w * FW, FW)])
            return 0

        lax.fori_loop(0, NFW, f_body, 0)

        def rk_body(t, _):
            ordv = ordout[pl.ds(t * LN, LN)]
            plsc.store_scatter(keys, [ordv], t * LN + iota)
            return 0

        lax.fori_loop(0, STR, rk_body, 0)
        pltpu.sync_copy(keys, rank_hbm.at[r])
        return carry0

    lax.fori_loop(0, RPW, do_row, 0)


def _mix_body(x_hbm, rank_hbm, svp_hbm, coef_hbm, out_hbm,
              svrow, xwin, rwin, owin, coefv):
    cid = lax.axis_index("c")
    sid = lax.axis_index("s")
    wid = sid * NC + cid

    def do_row(i, carry0):
        r = wid * RPW + i
        pltpu.sync_copy(svp_hbm.at[r], svrow)
        pltpu.sync_copy(coef_hbm.at[r], coefv)
        cf = coefv[...]

        def w_body(w, _):
            pltpu.sync_copy(x_hbm.at[r, pl.ds(w * FW, FW)], xwin)
            pltpu.sync_copy(rank_hbm.at[r, pl.ds(w * FW, FW)], rwin)

            def wi_body(tt, _):
                rk = rwin[pl.ds(tt * LN, LN)]
                xi = xwin[pl.ds(tt * LN, LN)]
                sva = plsc.load_gather(svrow, [rk])
                owin[pl.ds(tt * LN, LN)] = xi + (sva - xi) * cf
                return 0

            lax.fori_loop(0, FW // LN, wi_body, 0)
            pltpu.sync_copy(owin, out_hbm.at[r, pl.ds(w * FW, FW)])
            return 0

        lax.fori_loop(0, NFW, w_body, 0)
        return carry0

    lax.fori_loop(0, RPW, do_row, 0)


_MESH = plsc.VectorSubcoreMesh(
    core_axis_name="c", subcore_axis_name="s", num_cores=NC, num_subcores=NS)
_CPARAMS = pltpu.CompilerParams(
    use_tc_tiling_on_sc=False, needs_layout_passes=False)

_sort_call = pl.kernel(
    _sort_body,
    out_type=(
        jax.ShapeDtypeStruct((R, N), jnp.float32),
        jax.ShapeDtypeStruct((R, N), jnp.int32),
        jax.ShapeDtypeStruct((R, LN, STR), jnp.int32),
    ),
    mesh=_MESH,
    scratch_types=[
        pltpu.VMEM((N,), jnp.int32),
        pltpu.VMEM((N,), jnp.int32),
        pltpu.VMEM((LN, CHK), jnp.int32),
        pltpu.VMEM((4096,), jnp.int32),
        pltpu.VMEM((FW,), jnp.float32),
    ],
    compiler_params=_CPARAMS,
)

_mix_call = pl.kernel(
    _mix_body,
    out_type=jax.ShapeDtypeStruct((R, N), jnp.float32),
    mesh=_MESH,
    scratch_types=[
        pltpu.VMEM((N,), jnp.float32),
        pltpu.VMEM((FW,), jnp.float32),
        pltpu.VMEM((FW,), jnp.int32),
        pltpu.VMEM((FW,), jnp.float32),
        pltpu.VMEM((LN,), jnp.float32),
    ],
    compiler_params=_CPARAMS,
)


def kernel(x, lmda, perm):
    xr = x.reshape(R, N)
    xbits = lax.bitcast_convert_type(xr, jnp.int32)
    sv, rank, _ = _sort_call(xbits)
    svp = sv.reshape(B, C * N)[perm].reshape(R, N)
    coef = jnp.repeat(1.0 - lmda.reshape(B), C)
    coef16 = jnp.broadcast_to(coef[:, None], (R, LN)).astype(jnp.float32)
    out = _mix_call(xr, rank, svp, coef16)
    return out.reshape(B, C, H, W)


# speedup vs baseline: 1.0213x; 1.0213x over previous
import jax
import jax.numpy as jnp
import numpy as np
from jax import lax
from jax.experimental import pallas as pl
from jax.experimental.pallas import tpu as pltpu
from jax.experimental.pallas import tpu_sc as plsc

NC = 2
NS = 16
LN = 16
NW = NC * NS

B, C, H, W = 8, 96, 224, 224
N = H * W
R = B * C
RPW = R // NW
STR = N // LN
CHK = 448
NWIN = STR // CHK
FW = 1792
NFW = N // FW

H2 = 4096

MINI32 = np.int32(-2147483648)


def _sort_body(x_hbm, sv_hbm, rank_hbm, ords_hbm, keys, ordout, ordwin, hist,
               fwin):
    cid = lax.axis_index("c")
    sid = lax.axis_index("s")
    wid = sid * NC + cid
    iota = lax.iota(jnp.int32, LN)
    lane_base = iota * STR
    ones = jnp.ones((LN,), jnp.int32)
    zeros = jnp.zeros((LN,), jnp.int32)

    def do_row(i, carry0):
        r = wid * RPW + i
        pltpu.sync_copy(x_hbm.at[r], keys)

        def tf_body(t, _):
            for u in range(8):
                off = (t * 8 + u) * LN
                k = keys[pl.ds(off, LN)]
                keys[pl.ds(off, LN)] = k ^ ((k >> 31) | MINI32)
            return 0

        lax.fori_loop(0, STR // 8, tf_body, 0)

        for p in range(4):
            shift = 8 * p

            def z_body(h, _):
                for u in range(8):
                    hist[pl.ds((h * 8 + u) * LN, LN)] = zeros
                return 0

            lax.fori_loop(0, 2 * H2 // (8 * LN), z_body, 0)

            if p == 0:
                def c_body(t, _):
                    for u in range(4):
                        ordv = lane_base + (t * 4 + u)
                        k = plsc.load_gather(keys, [ordv])
                        d = (k >> shift) & 255
                        hidx = (d << 4) | iota
                        plsc.addupdate_scatter(hist, [hidx + (u % 2) * H2],
                                               ones)
                    return 0

                lax.fori_loop(0, STR // 4, c_body, 0)
            else:
                def cw_body(w, _):
                    pltpu.sync_copy(
                        ords_hbm.at[r, :, pl.ds(w * CHK, CHK)], ordwin)

                    def ci_body(tt, _):
                        for u in range(4):
                            col = iota * 0 + (tt * 4 + u)
                            ordv = plsc.load_gather(ordwin, [iota, col])
                            k = plsc.load_gather(keys, [ordv])
                            d = (k >> shift) & 255
                            hidx = (d << 4) | iota
                            plsc.addupdate_scatter(
                                hist, [hidx + (u % 2) * H2], ones)
                        return 0

                    lax.fori_loop(0, CHK // 4, ci_body, 0)
                    return 0

                lax.fori_loop(0, NWIN, cw_body, 0)

            def pf_body(h, carry):
                c = carry
                for u in range(2):
                    hh = h * 2 + u
                    v = (hist[pl.ds(hh * LN, LN)]
                         + hist[pl.ds(H2 + hh * LN, LN)])
                    inc = plsc.cumsum(v)
                    hist[pl.ds(hh * LN, LN)] = inc - v + c
                    c = c + jnp.sum(v)
                return c

            lax.fori_loop(0, 128, pf_body, jnp.int32(0))

            if p == 0:
                def s_body(t, _):
                    for u in range(4):
                        ordv = lane_base + (t * 4 + u)
                        k = plsc.load_gather(keys, [ordv])
                        d = (k >> shift) & 255
                        hidx = (d << 4) | iota
                        pos = plsc.load_gather(hist, [hidx])
                        plsc.store_scatter(ordout, [pos], ordv)
                        plsc.store_scatter(hist, [hidx], pos + 1)
                    return 0

                lax.fori_loop(0, STR // 4, s_body, 0)
            else:
                def sw_body(w, _):
                    pltpu.sync_copy(
                        ords_hbm.at[r, :, pl.ds(w * CHK, CHK)], ordwin)

                    def si_body(tt, _):
                        for u in range(4):
                            col = iota * 0 + (tt * 4 + u)
                            ordv = plsc.load_gather(ordwin, [iota, col])
                            k = plsc.load_gather(keys, [ordv])
                            d = (k >> shift) & 255
                            hidx = (d << 4) | iota
                            pos = plsc.load_gather(hist, [hidx])
                            plsc.store_scatter(ordout, [pos], ordv)
                            plsc.store_scatter(hist, [hidx], pos + 1)
                        return 0

                    lax.fori_loop(0, CHK // 4, si_body, 0)
                    return 0

                lax.fori_loop(0, NWIN, sw_body, 0)

            if p < 3:
                for l in range(LN):
                    pltpu.sync_copy(ordout.at[pl.ds(l * STR, STR)],
                                    ords_hbm.at[r, l])

        def f_body(w, _):
            def fi_body(tq, _):
                for u in range(8):
                    tt = tq * 8 + u
                    ordv = ordout[pl.ds(w * FW + tt * LN, LN)]
                    k = plsc.load_gather(keys, [ordv])
                    k = k ^ (jnp.bitwise_not(k >> 31) | MINI32)
                    fwin[pl.ds(tt * LN, LN)] = plsc.bitcast(k, jnp.float32)
                return 0

            lax.fori_loop(0, FW // LN // 8, fi_body, 0)
            pltpu.sync_copy(fwin, sv_hbm.at[r, pl.ds(w * FW, FW)])
            return 0

        lax.fori_loop(0, NFW, f_body, 0)

        def rk_body(t, _):
            for u in range(8):
                tt = t * 8 + u
                ordv = ordout[pl.ds(tt * LN, LN)]
                plsc.store_scatter(keys, [ordv], tt * LN + iota)
            return 0

        lax.fori_loop(0, STR // 8, rk_body, 0)
        pltpu.sync_copy(keys, rank_hbm.at[r])
        return carry0

    lax.fori_loop(0, RPW, do_row, 0)


def _mix_body(x_hbm, rank_hbm, svp_hbm, coef_hbm, out_hbm,
              svrow, xwin, rwin, owin, coefv):
    cid = lax.axis_index("c")
    sid = lax.axis_index("s")
    wid = sid * NC + cid

    def do_row(i, carry0):
        r = wid * RPW + i
        pltpu.sync_copy(svp_hbm.at[r], svrow)
        pltpu.sync_copy(coef_hbm.at[r], coefv)
        cf = coefv[...]

        def w_body(w, _):
            pltpu.sync_copy(x_hbm.at[r, pl.ds(w * FW, FW)], xwin)
            pltpu.sync_copy(rank_hbm.at[r, pl.ds(w * FW, FW)], rwin)

            def wi_body(tq, _):
                for u in range(8):
                    tt = tq * 8 + u
                    rk = rwin[pl.ds(tt * LN, LN)]
                    xi = xwin[pl.ds(tt * LN, LN)]
                    sva = plsc.load_gather(svrow, [rk])
                    owin[pl.ds(tt * LN, LN)] = xi + (sva - xi) * cf
                return 0

            lax.fori_loop(0, FW // LN // 8, wi_body, 0)
            pltpu.sync_copy(owin, out_hbm.at[r, pl.ds(w * FW, FW)])
            return 0

        lax.fori_loop(0, NFW, w_body, 0)
        return carry0

    lax.fori_loop(0, RPW, do_row, 0)


_MESH = plsc.VectorSubcoreMesh(
    core_axis_name="c", subcore_axis_name="s", num_cores=NC, num_subcores=NS)
_CPARAMS = pltpu.CompilerParams(
    use_tc_tiling_on_sc=False, needs_layout_passes=False)

_sort_call = pl.kernel(
    _sort_body,
    out_type=(
        jax.ShapeDtypeStruct((R, N), jnp.float32),
        jax.ShapeDtypeStruct((R, N), jnp.int32),
        jax.ShapeDtypeStruct((R, LN, STR), jnp.int32),
    ),
    mesh=_MESH,
    scratch_types=[
        pltpu.VMEM((N,), jnp.int32),
        pltpu.VMEM((N,), jnp.int32),
        pltpu.VMEM((LN, CHK), jnp.int32),
        pltpu.VMEM((2 * H2,), jnp.int32),
        pltpu.VMEM((FW,), jnp.float32),
    ],
    compiler_params=_CPARAMS,
)

_mix_call = pl.kernel(
    _mix_body,
    out_type=jax.ShapeDtypeStruct((R, N), jnp.float32),
    mesh=_MESH,
    scratch_types=[
        pltpu.VMEM((N,), jnp.float32),
        pltpu.VMEM((FW,), jnp.float32),
        pltpu.VMEM((FW,), jnp.int32),
        pltpu.VMEM((FW,), jnp.float32),
        pltpu.VMEM((LN,), jnp.float32),
    ],
    compiler_params=_CPARAMS,
)


def kernel(x, lmda, perm):
    xr = x.reshape(R, N)
    xbits = lax.bitcast_convert_type(xr, jnp.int32)
    sv, rank, _ = _sort_call(xbits)
    svp = sv.reshape(B, C * N)[perm].reshape(R, N)
    coef = jnp.repeat(1.0 - lmda.reshape(B), C)
    coef16 = jnp.broadcast_to(coef[:, None], (R, LN)).astype(jnp.float32)
    out = _mix_call(xr, rank, svp, coef16)
    return out.reshape(B, C, H, W)
